# baseline (speedup 1.0000x reference)
import functools
import math

import jax
import jax.numpy as jnp
from jax import lax
from jax.experimental import pallas as pl
from jax.experimental.pallas import tpu as pltpu
from jax.experimental.pallas import tpu_sc as plsc

N = 100000
D = 32
E = 1600000
NL = 5
NB = 100
NCLS = 10
HALF = N // 2
NS = 16
LANES = 16
SJ = 3
NSLOT = 2
GRP = SJ * 128
NG = 262
PT = NG * GRP
EPAD = NS * PT
ROWS = EPAD // 128
PTR = PT // 128
FL = 3128
FLL = HALF - (NS - 1) * FL
ACC_ROWS = NS * FL
DW = 16
BN_SCALE = 1.0 / math.sqrt(1.0 + 1e-5)

_sc_mesh = plsc.VectorSubcoreMesh(core_axis_name="c", subcore_axis_name="s")


def _dloc_compute(didx, dlocs, base, j):
    dl = dlocs[j]
    for k2 in range(128 // LANES):
        v = didx[j, pl.ds(k2 * LANES, LANES)]
        loc = v - base
        ok = (loc >= 0) & (loc < HALF)
        dl[pl.ds(k2 * LANES, LANES)] = jnp.where(ok, loc, HALF)


def _agg_body(h_hbm, src_hbm, dst_hbm, zeros_hbm, out_hbm,
              accum, sidx0, sidx1, didx0, didx1, rows0, rows1,
              dl00, dl01, dl02, dl10, dl11, dl12, isem, gsem, ssem):
    sidx = (sidx0, sidx1)
    didx = (didx0, didx1)
    rows = (rows0, rows1)
    dlocs = ((dl00, dl01, dl02), (dl10, dl11, dl12))
    cid = lax.axis_index("c")
    sid = lax.axis_index("s")
    base = cid * HALF
    tbase = sid * PTR

    pltpu.sync_copy(zeros_hbm, accum.at[pl.ds(sid * FL, FL)])
    plsc.subcore_barrier()

    for b in range(NSLOT):
        pltpu.async_copy(src_hbm.at[pl.ds(tbase + b * SJ, SJ)], sidx[b],
                         isem)
        pltpu.async_copy(dst_hbm.at[pl.ds(tbase + b * SJ, SJ)], didx[b],
                         isem)

    def outer(o, carry):
        for b in range(NSLOT):
            g = o * NSLOT + b
            @pl.when(o > 0)
            def _():
                for j in range(SJ):
                    pltpu.make_async_copy(rows[b].at[j],
                                          accum.at[dlocs[b][j]],
                                          ssem).wait()
            pltpu.make_async_copy(src_hbm.at[pl.ds(0, SJ)], sidx[b],
                                  isem).wait()
            pltpu.make_async_copy(dst_hbm.at[pl.ds(0, SJ)], didx[b],
                                  isem).wait()
            gets = [pltpu.async_copy(h_hbm.at[sidx[b].at[j]],
                                     rows[b].at[j], gsem)
                    for j in range(SJ)]
            for j in range(SJ):
                _dloc_compute(didx[b], dlocs[b], base, j)
            for cp in gets:
                cp.wait()
            for j in range(SJ):
                pltpu.async_copy(rows[b].at[j], accum.at[dlocs[b][j]],
                                 ssem, add=True)
            rbn = tbase + jnp.minimum((g + NSLOT) * SJ, PTR - SJ)
            pltpu.async_copy(src_hbm.at[pl.ds(rbn, SJ)], sidx[b], isem)
            pltpu.async_copy(dst_hbm.at[pl.ds(rbn, SJ)], didx[b], isem)
        return carry

    lax.fori_loop(0, NG // NSLOT, outer, 0)
    for b in range(NSLOT):
        for j in range(SJ):
            pltpu.make_async_copy(rows[b].at[j], accum.at[dlocs[b][j]],
                                  ssem).wait()
        pltpu.make_async_copy(src_hbm.at[pl.ds(0, SJ)], sidx[b],
                              isem).wait()
        pltpu.make_async_copy(dst_hbm.at[pl.ds(0, SJ)], didx[b],
                              isem).wait()
    plsc.subcore_barrier()

    @pl.when(sid < NS - 1)
    def _():
        pltpu.sync_copy(accum.at[pl.ds(sid * FL, FL)],
                        out_hbm.at[pl.ds(base + sid * FL, FL)])

    @pl.when(sid == NS - 1)
    def _():
        pltpu.sync_copy(accum.at[pl.ds((NS - 1) * FL, FLL)],
                        out_hbm.at[pl.ds(base + (NS - 1) * FL, FLL)])


_agg_call = functools.partial(
    pl.kernel,
    out_type=jax.ShapeDtypeStruct((N, D), jnp.float32),
    mesh=_sc_mesh,
    compiler_params=pltpu.CompilerParams(use_tc_tiling_on_sc=False),
    scratch_types=[
        pltpu.VMEM_SHARED((ACC_ROWS, D), jnp.float32),
    ] + [pltpu.VMEM((SJ, 128), jnp.int32) for _ in range(2 * NSLOT)] + [
        pltpu.VMEM((SJ, 128, D), jnp.float32) for _ in range(NSLOT)
    ] + [pltpu.VMEM((128,), jnp.int32) for _ in range(NSLOT * SJ)] + [
        pltpu.SemaphoreType.DMA,
        pltpu.SemaphoreType.DMA,
        pltpu.SemaphoreType.DMA,
    ],
)(_agg_body)


def _deg_body(dst_hbm, ones_hbm, zeros_hbm, out_hbm,
              accum, onesv, didx0, didx1,
              dl00, dl01, dl02, dl10, dl11, dl12, isem, ssem):
    didx = (didx0, didx1)
    dlocs = ((dl00, dl01, dl02), (dl10, dl11, dl12))
    cid = lax.axis_index("c")
    sid = lax.axis_index("s")
    base = cid * HALF
    tbase = sid * PTR

    pltpu.sync_copy(zeros_hbm, accum.at[pl.ds(sid * FL, FL)])
    pltpu.sync_copy(ones_hbm, onesv)
    plsc.subcore_barrier()

    for b in range(NSLOT):
        pltpu.async_copy(dst_hbm.at[pl.ds(tbase + b * SJ, SJ)], didx[b],
                         isem)

    def outer(o, carry):
        for b in range(NSLOT):
            g = o * NSLOT + b
            @pl.when(o > 0)
            def _():
                for j in range(SJ):
                    pltpu.make_async_copy(onesv, accum.at[dlocs[b][j]],
                                          ssem).wait()
            pltpu.make_async_copy(dst_hbm.at[pl.ds(0, SJ)], didx[b],
                                  isem).wait()
            for j in range(SJ):
                _dloc_compute(didx[b], dlocs[b], base, j)
            for j in range(SJ):
                pltpu.async_copy(onesv, accum.at[dlocs[b][j]], ssem,
                                 add=True)
            rbn = tbase + jnp.minimum((g + NSLOT) * SJ, PTR - SJ)
            pltpu.async_copy(dst_hbm.at[pl.ds(rbn, SJ)], didx[b], isem)
        return carry

    lax.fori_loop(0, NG // NSLOT, outer, 0)
    for b in range(NSLOT):
        for j in range(SJ):
            pltpu.make_async_copy(onesv, accum.at[dlocs[b][j]],
                                  ssem).wait()
        pltpu.make_async_copy(dst_hbm.at[pl.ds(0, SJ)], didx[b],
                              isem).wait()
    plsc.subcore_barrier()

    @pl.when(sid < NS - 1)
    def _():
        pltpu.sync_copy(accum.at[pl.ds(sid * FL, FL)],
                        out_hbm.at[pl.ds(base + sid * FL, FL)])

    @pl.when(sid == NS - 1)
    def _():
        pltpu.sync_copy(accum.at[pl.ds((NS - 1) * FL, FLL)],
                        out_hbm.at[pl.ds(base + (NS - 1) * FL, FLL)])


_deg_call = functools.partial(
    pl.kernel,
    out_type=jax.ShapeDtypeStruct((N, DW), jnp.float32),
    mesh=_sc_mesh,
    compiler_params=pltpu.CompilerParams(use_tc_tiling_on_sc=False),
    scratch_types=[
        pltpu.VMEM_SHARED((ACC_ROWS, DW), jnp.float32),
        pltpu.VMEM((128, DW), jnp.float32),
    ] + [pltpu.VMEM((SJ, 128), jnp.int32) for _ in range(NSLOT)] + [
        pltpu.VMEM((128,), jnp.int32) for _ in range(NSLOT * SJ)
    ] + [
        pltpu.SemaphoreType.DMA,
        pltpu.SemaphoreType.DMA,
    ],
)(_deg_body)


BLK = 2000


def _dense_body(h_ref, agg_ref, deg_ref, w1_ref, b1_ref, w2_ref, b2_ref,
                gm_ref, bt_ref, o_ref):
    deg = jnp.maximum(deg_ref[:, 0:1], 1.0)
    rst = h_ref[...] + agg_ref[...] / deg
    u = jnp.maximum(
        jnp.dot(rst, w1_ref[...], preferred_element_type=jnp.float32)
        + b1_ref[...], 0.0)
    y = jnp.dot(u, w2_ref[...], preferred_element_type=jnp.float32) \
        + b2_ref[...]
    o_ref[...] = gm_ref[...] * (jnp.maximum(y, 0.0) * BN_SCALE) + bt_ref[...]


_dense_call = pl.pallas_call(
    _dense_body,
    grid=(N // BLK,),
    in_specs=[
        pl.BlockSpec((BLK, D), lambda i: (i, 0)),
        pl.BlockSpec((BLK, D), lambda i: (i, 0)),
        pl.BlockSpec((BLK, DW), lambda i: (i, 0)),
        pl.BlockSpec((D, D), lambda i: (0, 0)),
        pl.BlockSpec((1, D), lambda i: (0, 0)),
        pl.BlockSpec((D, D), lambda i: (0, 0)),
        pl.BlockSpec((1, D), lambda i: (0, 0)),
        pl.BlockSpec((1, D), lambda i: (0, 0)),
        pl.BlockSpec((1, D), lambda i: (0, 0)),
    ],
    out_specs=pl.BlockSpec((BLK, D), lambda i: (i, 0)),
    out_shape=jax.ShapeDtypeStruct((N, D), jnp.float32),
)


def _gather_body(idx_ref, h_ref, o_ref):
    o_ref[...] = h_ref[...]


_gather_call = pl.pallas_call(
    _gather_body,
    grid_spec=pltpu.PrefetchScalarGridSpec(
        num_scalar_prefetch=1,
        grid=(NB,),
        in_specs=[pl.BlockSpec((1, 1, D), lambda i, idx: (idx[i], 0, 0))],
        out_specs=pl.BlockSpec((1, 1, D), lambda i, idx: (i, 0, 0)),
    ),
    out_shape=jax.ShapeDtypeStruct((NB, 1, D), jnp.float32),
)


def _head_body(g_ref, w1_ref, b1_ref, w2_ref, b2_ref, o_ref):
    g1 = jnp.maximum(
        jnp.dot(g_ref[...], w1_ref[...], preferred_element_type=jnp.float32)
        + b1_ref[...], 0.0)
    logits = jnp.dot(g1, w2_ref[...], preferred_element_type=jnp.float32) \
        + b2_ref[...]
    m = jnp.max(logits, axis=-1, keepdims=True)
    lse = jnp.log(jnp.sum(jnp.exp(logits - m), axis=-1, keepdims=True)) + m
    o_ref[...] = logits - lse


_head_call = pl.pallas_call(
    _head_body,
    out_shape=jax.ShapeDtypeStruct((NB, NCLS), jnp.float32),
)


def kernel(x, edge_index, batch_num_nodes, W1, b1, W2, b2, gamma, beta,
           fc1_w, fc1_b, fc2_w, fc2_b):
    src = edge_index[0]
    dst = edge_index[1]
    pad = EPAD - E
    src2 = jnp.concatenate(
        [src, jnp.zeros((pad,), jnp.int32)]).reshape(ROWS, 128)
    dst2 = jnp.concatenate(
        [dst, jnp.full((pad,), -1, jnp.int32)]).reshape(ROWS, 128)
    zeros32 = jnp.zeros((FL, D), jnp.float32)
    zeros16 = jnp.zeros((FL, DW), jnp.float32)
    ones16 = jnp.ones((128, DW), jnp.float32)

    degf = _deg_call(dst2, ones16, zeros16)

    h = x
    for i in range(NL):
        agg = _agg_call(h, src2, dst2, zeros32)
        h = _dense_call(h, agg, degf, W1[i], b1[i].reshape(1, D), W2[i],
                        b2[i].reshape(1, D), gamma[i].reshape(1, D),
                        beta[i].reshape(1, D))

    idx = (jnp.cumsum(batch_num_nodes) - 1).astype(jnp.int32)
    g = _gather_call(idx, h.reshape(N, 1, D)).reshape(NB, D)
    return _head_call(g, fc1_w, fc1_b.reshape(1, D), fc2_w,
                      fc2_b.reshape(1, NCLS))


# speedup vs baseline: 2.4422x; 2.4422x over previous
import functools
import math

import jax
import jax.numpy as jnp
from jax import lax
from jax.experimental import pallas as pl
from jax.experimental.pallas import tpu as pltpu
from jax.experimental.pallas import tpu_sc as plsc

N = 100000
D = 32
E = 1600000
NL = 5
NB = 100
NCLS = 10
HALF = N // 2
NC = 2
NS = 16
LANES = 16

NW = NC * NS
WPR = 396
ROWS = NW * WPR
EPAD = ROWS * 128
CR = 3
NCH = WPR // CR
CAPR = 396
CAPF = CAPR * 128
FRE = 4096
BUFCAP = 5376

SJ = 3
NSLOT = 2
GRP = SJ * 128

FL = 3128
FLL = HALF - (NS - 1) * FL
ACC_ROWS = NS * FL
DW = 16
BN_SCALE = 1.0 / math.sqrt(1.0 + 1e-5)

_sc_mesh = plsc.VectorSubcoreMesh(core_axis_name="c", subcore_axis_name="s")
_sc_params = pltpu.CompilerParams(use_tc_tiling_on_sc=False,
                                  needs_layout_passes=False)


def _part_body(src_hbm, dst_hbm, psrc_hbm, pdst_hbm, cnt_hbm,
               sidx0, sidx1, didx0, didx1,
               sbuf0, dbuf0, sbuf1, dbuf1, stgs, stgd, cntw, isem):
    sidx = (sidx0, sidx1)
    didx = (didx0, didx1)
    sbufs = (sbuf0, sbuf1)
    dbufs = (dbuf0, dbuf1)
    cid = lax.axis_index("c")
    sid = lax.axis_index("s")
    wid = cid * NS + sid
    inbase = wid * WPR

    for b in range(NSLOT):
        pltpu.async_copy(src_hbm.at[pl.ds(inbase + b * CR, CR)], sidx[b],
                         isem)
        pltpu.async_copy(dst_hbm.at[pl.ds(inbase + b * CR, CR)], didx[b],
                         isem)

    def outer(o, st):
        off0, wofs0, off1, wofs1 = st
        offs = [off0, off1]
        wofs = [wofs0, wofs1]
        for b in range(NSLOT):
            ch = o * NSLOT + b
            pltpu.make_async_copy(src_hbm.at[pl.ds(0, CR)], sidx[b],
                                  isem).wait()
            pltpu.make_async_copy(dst_hbm.at[pl.ds(0, CR)], didx[b],
                                  isem).wait()
            for j in range(CR):
                for k2 in range(128 // LANES):
                    s_v = sidx[b][j, pl.ds(k2 * LANES, LANES)]
                    d_v = didx[b][j, pl.ds(k2 * LANES, LANES)]
                    for h in range(2):
                        if h == 0:
                            m = (d_v >= 0) & (d_v < HALF)
                            dl = d_v
                        else:
                            m = d_v >= HALF
                            dl = d_v - HALF
                        plsc.store_compressed(stgs.at[pl.ds(0, LANES)],
                                              s_v, mask=m)
                        plsc.store_compressed(stgd.at[pl.ds(0, LANES)],
                                              dl, mask=m)
                        c = jnp.max(plsc.all_reduce_population_count(m))
                        sbufs[h][pl.ds(offs[h], LANES)] = stgs[...]
                        dbufs[h][pl.ds(offs[h], LANES)] = stgd[...]
                        offs[h] = offs[h] + c
            nofs = inbase + jnp.minimum((ch + NSLOT) * CR, WPR - CR)
            pltpu.async_copy(src_hbm.at[pl.ds(nofs, CR)], sidx[b], isem)
            pltpu.async_copy(dst_hbm.at[pl.ds(nofs, CR)], didx[b], isem)
        for h in range(2):
            off = offs[h]
            wo = wofs[h]

            @pl.when(off >= FRE)
            def _(h=h, wo=wo):
                woa = pl.multiple_of(wo, 128)
                pltpu.sync_copy(sbufs[h].at[pl.ds(0, FRE)],
                                psrc_hbm.at[h, wid, pl.ds(woa, FRE)])
                pltpu.sync_copy(dbufs[h].at[pl.ds(0, FRE)],
                                pdst_hbm.at[h, wid, pl.ds(woa, FRE)])
                for t in range(24):
                    rv = sbufs[h][pl.ds(FRE + t * LANES, LANES)]
                    sbufs[h][pl.ds(t * LANES, LANES)] = rv
                    rv2 = dbufs[h][pl.ds(FRE + t * LANES, LANES)]
                    dbufs[h][pl.ds(t * LANES, LANES)] = rv2

            offs[h] = jnp.where(off >= FRE, off - FRE, off)
            wofs[h] = jnp.where(off >= FRE, wo + FRE, wo)
        return offs[0], wofs[0], offs[1], wofs[1]

    z = jnp.int32(0)
    off0, wofs0, off1, wofs1 = lax.fori_loop(0, NCH // NSLOT, outer,
                                             (z, z, z, z))

    for b in range(NSLOT):
        pltpu.make_async_copy(src_hbm.at[pl.ds(0, CR)], sidx[b],
                              isem).wait()
        pltpu.make_async_copy(dst_hbm.at[pl.ds(0, CR)], didx[b],
                              isem).wait()

    for h in range(2):
        off = off0 if h == 0 else off1
        wo = wofs0 if h == 0 else wofs1
        total = wo + off
        unit = SJ * NSLOT * 128
        nunits = (total + unit - 1) // unit
        target = nunits * unit
        for t in range(48):
            sbufs[h][pl.ds(off + t * LANES, LANES)] = jnp.zeros(
                (LANES,), jnp.int32)
            dbufs[h][pl.ds(off + t * LANES, LANES)] = jnp.full(
                (LANES,), HALF, jnp.int32)

        def tail(r, wo_, h=h):
            ta = pl.multiple_of(wo_ + r * 128, 128)
            pltpu.sync_copy(sbufs[h].at[pl.ds(r * 128, 128)],
                            psrc_hbm.at[h, wid, pl.ds(ta, 128)])
            pltpu.sync_copy(dbufs[h].at[pl.ds(r * 128, 128)],
                            pdst_hbm.at[h, wid, pl.ds(ta, 128)])
            return wo_

        lax.fori_loop(0, (target - wo) // 128, tail, wo)
        cntw[pl.ds(0, LANES)] = jnp.full((LANES,), nunits, jnp.int32)
        pltpu.sync_copy(cntw, cnt_hbm.at[h, wid])


_part_call = functools.partial(
    pl.kernel,
    out_type=(
        jax.ShapeDtypeStruct((2, NW, CAPF), jnp.int32),
        jax.ShapeDtypeStruct((2, NW, CAPF), jnp.int32),
        jax.ShapeDtypeStruct((2, NW, LANES), jnp.int32),
    ),
    mesh=_sc_mesh,
    compiler_params=_sc_params,
    scratch_types=[
        pltpu.VMEM((CR, 128), jnp.int32) for _ in range(2 * NSLOT)
    ] + [pltpu.VMEM((BUFCAP,), jnp.int32) for _ in range(4)] + [
        pltpu.VMEM((LANES,), jnp.int32),
        pltpu.VMEM((LANES,), jnp.int32),
        pltpu.VMEM((LANES,), jnp.int32),
        pltpu.SemaphoreType.DMA,
    ],
)(_part_body)


def _edge_pass(gather, h_hbm, psrc_hbm, pdst_hbm, cnt_hbm, accum, sidx,
               didx, rows, dlocs, onesv, cntv, isem, gsem, ssem, cid, sid):
    for r in range(2):
        w = 2 * sid + r
        pltpu.sync_copy(cnt_hbm.at[cid, w], cntv)
        no = jnp.max(cntv[...])
        ngrp = no * NSLOT
        for b in range(NSLOT):
            pltpu.async_copy(psrc_hbm.at[cid, w, pl.ds(b * GRP, GRP)],
                             sidx[b], isem)
            pltpu.async_copy(pdst_hbm.at[cid, w, pl.ds(b * GRP, GRP)],
                             didx[b], isem)

        def outer(o, carry, w=w, ngrp=ngrp):
            for b in range(NSLOT):
                g = o * NSLOT + b

                @pl.when(o > 0)
                def _():
                    for j in range(SJ):
                        src = rows[b].at[j] if gather else onesv
                        pltpu.make_async_copy(src, accum.at[dlocs[b][j]],
                                              ssem).wait()

                pltpu.make_async_copy(psrc_hbm.at[cid, w, pl.ds(0, GRP)],
                                      sidx[b], isem).wait()
                pltpu.make_async_copy(pdst_hbm.at[cid, w, pl.ds(0, GRP)],
                                      didx[b], isem).wait()
                if gather:
                    gets = [pltpu.async_copy(
                        h_hbm.at[sidx[b].at[pl.ds(j * 128, 128)]],
                        rows[b].at[j], gsem) for j in range(SJ)]
                for j in range(SJ):
                    for k2 in range(128 // LANES):
                        dlocs[b][j][pl.ds(k2 * LANES, LANES)] = \
                            didx[b][pl.ds(j * 128 + k2 * LANES, LANES)]
                if gather:
                    for cp in gets:
                        cp.wait()
                for j in range(SJ):
                    src = rows[b].at[j] if gather else onesv
                    pltpu.async_copy(src, accum.at[dlocs[b][j]], ssem,
                                     add=True)
                nofs = pl.multiple_of(
                    jnp.minimum(g + NSLOT, ngrp - 1) * GRP, 128)
                pltpu.async_copy(psrc_hbm.at[cid, w, pl.ds(nofs, GRP)],
                                 sidx[b], isem)
                pltpu.async_copy(pdst_hbm.at[cid, w, pl.ds(nofs, GRP)],
                                 didx[b], isem)
            return carry

        lax.fori_loop(0, no, outer, 0)

        @pl.when(no > 0)
        def _():
            for b in range(NSLOT):
                for j in range(SJ):
                    src = rows[b].at[j] if gather else onesv
                    pltpu.make_async_copy(src, accum.at[dlocs[b][j]],
                                          ssem).wait()
        for b in range(NSLOT):
            pltpu.make_async_copy(psrc_hbm.at[cid, 0, pl.ds(0, GRP)],
                                  sidx[b], isem).wait()
            pltpu.make_async_copy(pdst_hbm.at[cid, 0, pl.ds(0, GRP)],
                                  didx[b], isem).wait()


def _flush(accum, out_hbm, cid, sid):
    base = cid * HALF

    @pl.when(sid < NS - 1)
    def _():
        pltpu.sync_copy(accum.at[pl.ds(sid * FL, FL)],
                        out_hbm.at[pl.ds(base + sid * FL, FL)])

    @pl.when(sid == NS - 1)
    def _():
        pltpu.sync_copy(accum.at[pl.ds((NS - 1) * FL, FLL)],
                        out_hbm.at[pl.ds(base + (NS - 1) * FL, FLL)])


def _agg_body(h_hbm, psrc_hbm, pdst_hbm, cnt_hbm, zeros_hbm, out_hbm,
              accum, sidx0, sidx1, didx0, didx1, rows0, rows1,
              dl00, dl01, dl02, dl10, dl11, dl12, cntv, isem, gsem, ssem):
    cid = lax.axis_index("c")
    sid = lax.axis_index("s")
    pltpu.sync_copy(zeros_hbm, accum.at[pl.ds(sid * FL, FL)])
    plsc.subcore_barrier()
    _edge_pass(True, h_hbm, psrc_hbm, pdst_hbm, cnt_hbm, accum,
               (sidx0, sidx1), (didx0, didx1), (rows0, rows1),
               ((dl00, dl01, dl02), (dl10, dl11, dl12)), None,
               cntv, isem, gsem, ssem, cid, sid)
    plsc.subcore_barrier()
    _flush(accum, out_hbm, cid, sid)


_agg_call = functools.partial(
    pl.kernel,
    out_type=jax.ShapeDtypeStruct((N, D), jnp.float32),
    mesh=_sc_mesh,
    compiler_params=_sc_params,
    scratch_types=[
        pltpu.VMEM_SHARED((ACC_ROWS, D), jnp.float32),
    ] + [pltpu.VMEM((GRP,), jnp.int32) for _ in range(2 * NSLOT)] + [
        pltpu.VMEM((SJ, 128, D), jnp.float32) for _ in range(NSLOT)
    ] + [pltpu.VMEM((128,), jnp.int32) for _ in range(NSLOT * SJ)] + [
        pltpu.VMEM((LANES,), jnp.int32),
        pltpu.SemaphoreType.DMA,
        pltpu.SemaphoreType.DMA,
        pltpu.SemaphoreType.DMA,
    ],
)(_agg_body)


def _deg_body(psrc_hbm, pdst_hbm, cnt_hbm, ones_hbm, zeros_hbm, out_hbm,
              accum, onesv, sidx0, sidx1, didx0, didx1,
              dl00, dl01, dl02, dl10, dl11, dl12, cntv, isem, gsem, ssem):
    cid = lax.axis_index("c")
    sid = lax.axis_index("s")
    pltpu.sync_copy(zeros_hbm, accum.at[pl.ds(sid * FL, FL)])
    pltpu.sync_copy(ones_hbm, onesv)
    plsc.subcore_barrier()
    _edge_pass(False, None, psrc_hbm, pdst_hbm, cnt_hbm, accum,
               (sidx0, sidx1), (didx0, didx1), None,
               ((dl00, dl01, dl02), (dl10, dl11, dl12)), onesv,
               cntv, isem, gsem, ssem, cid, sid)
    plsc.subcore_barrier()
    _flush(accum, out_hbm, cid, sid)


_deg_call = functools.partial(
    pl.kernel,
    out_type=jax.ShapeDtypeStruct((N, DW), jnp.float32),
    mesh=_sc_mesh,
    compiler_params=_sc_params,
    scratch_types=[
        pltpu.VMEM_SHARED((ACC_ROWS, DW), jnp.float32),
        pltpu.VMEM((128, DW), jnp.float32),
    ] + [pltpu.VMEM((GRP,), jnp.int32) for _ in range(2 * NSLOT)] + [
        pltpu.VMEM((128,), jnp.int32) for _ in range(NSLOT * SJ)
    ] + [
        pltpu.VMEM((LANES,), jnp.int32),
        pltpu.SemaphoreType.DMA,
        pltpu.SemaphoreType.DMA,
        pltpu.SemaphoreType.DMA,
    ],
)(_deg_body)


BLK = 2000


def _dense_body(h_ref, agg_ref, deg_ref, w1_ref, b1_ref, w2_ref, b2_ref,
                gm_ref, bt_ref, o_ref):
    deg = jnp.maximum(deg_ref[:, 0:1], 1.0)
    rst = h_ref[...] + agg_ref[...] / deg
    u = jnp.maximum(
        jnp.dot(rst, w1_ref[...], preferred_element_type=jnp.float32)
        + b1_ref[...], 0.0)
    y = jnp.dot(u, w2_ref[...], preferred_element_type=jnp.float32) \
        + b2_ref[...]
    o_ref[...] = gm_ref[...] * (jnp.maximum(y, 0.0) * BN_SCALE) + bt_ref[...]


_dense_call = pl.pallas_call(
    _dense_body,
    grid=(N // BLK,),
    in_specs=[
        pl.BlockSpec((BLK, D), lambda i: (i, 0)),
        pl.BlockSpec((BLK, D), lambda i: (i, 0)),
        pl.BlockSpec((BLK, DW), lambda i: (i, 0)),
        pl.BlockSpec((D, D), lambda i: (0, 0)),
        pl.BlockSpec((1, D), lambda i: (0, 0)),
        pl.BlockSpec((D, D), lambda i: (0, 0)),
        pl.BlockSpec((1, D), lambda i: (0, 0)),
        pl.BlockSpec((1, D), lambda i: (0, 0)),
        pl.BlockSpec((1, D), lambda i: (0, 0)),
    ],
    out_specs=pl.BlockSpec((BLK, D), lambda i: (i, 0)),
    out_shape=jax.ShapeDtypeStruct((N, D), jnp.float32),
)


def _gather_body(idx_ref, h_ref, o_ref):
    o_ref[...] = h_ref[...]


_gather_call = pl.pallas_call(
    _gather_body,
    grid_spec=pltpu.PrefetchScalarGridSpec(
        num_scalar_prefetch=1,
        grid=(NB,),
        in_specs=[pl.BlockSpec((1, 1, D), lambda i, idx: (idx[i], 0, 0))],
        out_specs=pl.BlockSpec((1, 1, D), lambda i, idx: (i, 0, 0)),
    ),
    out_shape=jax.ShapeDtypeStruct((NB, 1, D), jnp.float32),
)


def _head_body(g_ref, w1_ref, b1_ref, w2_ref, b2_ref, o_ref):
    g1 = jnp.maximum(
        jnp.dot(g_ref[...], w1_ref[...], preferred_element_type=jnp.float32)
        + b1_ref[...], 0.0)
    logits = jnp.dot(g1, w2_ref[...], preferred_element_type=jnp.float32) \
        + b2_ref[...]
    m = jnp.max(logits, axis=-1, keepdims=True)
    lse = jnp.log(jnp.sum(jnp.exp(logits - m), axis=-1, keepdims=True)) + m
    o_ref[...] = logits - lse


_head_call = pl.pallas_call(
    _head_body,
    out_shape=jax.ShapeDtypeStruct((NB, NCLS), jnp.float32),
)


def kernel(x, edge_index, batch_num_nodes, W1, b1, W2, b2, gamma, beta,
           fc1_w, fc1_b, fc2_w, fc2_b):
    src = edge_index[0]
    dst = edge_index[1]
    pad = EPAD - E
    src2 = jnp.concatenate(
        [src, jnp.zeros((pad,), jnp.int32)]).reshape(ROWS, 128)
    dst2 = jnp.concatenate(
        [dst, jnp.full((pad,), -1, jnp.int32)]).reshape(ROWS, 128)
    zeros32 = jnp.zeros((FL, D), jnp.float32)
    zeros16 = jnp.zeros((FL, DW), jnp.float32)
    ones16 = jnp.ones((128, DW), jnp.float32)

    psrc, pdst, cnt = _part_call(src2, dst2)
    degf = _deg_call(psrc, pdst, cnt, ones16, zeros16)

    h = x
    for i in range(NL):
        agg = _agg_call(h, psrc, pdst, cnt, zeros32)
        h = _dense_call(h, agg, degf, W1[i], b1[i].reshape(1, D), W2[i],
                        b2[i].reshape(1, D), gamma[i].reshape(1, D),
                        beta[i].reshape(1, D))

    idx = (jnp.cumsum(batch_num_nodes) - 1).astype(jnp.int32)
    g = _gather_call(idx, h.reshape(N, 1, D)).reshape(NB, D)
    return _head_call(g, fc1_w, fc1_b.reshape(1, D), fc2_w,
                      fc2_b.reshape(1, NCLS))


# speedup vs baseline: 2.9741x; 1.2178x over previous
import functools
import math

import jax
import jax.numpy as jnp
from jax import lax
from jax.experimental import pallas as pl
from jax.experimental.pallas import tpu as pltpu
from jax.experimental.pallas import tpu_sc as plsc

N = 100000
D = 32
E = 1600000
NL = 5
NB = 100
NCLS = 10
HALF = N // 2
NC = 2
NS = 16
LANES = 16

NW = NC * NS
WPR = 396
ROWS = NW * WPR
EPAD = ROWS * 128
CR = 3
NCH = WPR // CR
CAPR = 396
CAPF = CAPR * 128
FRE = 4096
BUFCAP = 5376

SJ = 3
NSLOT = 2
GRP = SJ * 128

FL = 3128
FLL = HALF - (NS - 1) * FL
ACC_ROWS = NS * FL
BN_SCALE = 1.0 / math.sqrt(1.0 + 1e-5)

_sc_mesh = plsc.VectorSubcoreMesh(core_axis_name="c", subcore_axis_name="s")
_sc_params = pltpu.CompilerParams(use_tc_tiling_on_sc=False,
                                  needs_layout_passes=False)


def _part_body(src_hbm, dst_hbm, psrc_hbm, pdst_hbm, cnt_hbm,
               sidx0, sidx1, didx0, didx1,
               sbuf0, dbuf0, sbuf1, dbuf1, stgs, stgd, cntw, isem):
    sidx = (sidx0, sidx1)
    didx = (didx0, didx1)
    sbufs = (sbuf0, sbuf1)
    dbufs = (dbuf0, dbuf1)
    cid = lax.axis_index("c")
    sid = lax.axis_index("s")
    wid = cid * NS + sid
    inbase = wid * WPR

    for b in range(NSLOT):
        pltpu.async_copy(src_hbm.at[pl.ds(inbase + b * CR, CR)], sidx[b],
                         isem)
        pltpu.async_copy(dst_hbm.at[pl.ds(inbase + b * CR, CR)], didx[b],
                         isem)

    def outer(o, st):
        off0, wofs0, off1, wofs1 = st
        offs = [off0, off1]
        wofs = [wofs0, wofs1]
        for b in range(NSLOT):
            ch = o * NSLOT + b
            pltpu.make_async_copy(src_hbm.at[pl.ds(0, CR)], sidx[b],
                                  isem).wait()
            pltpu.make_async_copy(dst_hbm.at[pl.ds(0, CR)], didx[b],
                                  isem).wait()
            for j in range(CR):
                for k2 in range(128 // LANES):
                    s_v = sidx[b][j, pl.ds(k2 * LANES, LANES)]
                    d_v = didx[b][j, pl.ds(k2 * LANES, LANES)]
                    for h in range(2):
                        if h == 0:
                            m = (d_v >= 0) & (d_v < HALF)
                            dl = d_v
                        else:
                            m = d_v >= HALF
                            dl = d_v - HALF
                        plsc.store_compressed(stgs.at[pl.ds(0, LANES)],
                                              s_v, mask=m)
                        plsc.store_compressed(stgd.at[pl.ds(0, LANES)],
                                              dl, mask=m)
                        c = jnp.max(plsc.all_reduce_population_count(m))
                        sbufs[h][pl.ds(offs[h], LANES)] = stgs[...]
                        dbufs[h][pl.ds(offs[h], LANES)] = stgd[...]
                        offs[h] = offs[h] + c
            nofs = inbase + jnp.minimum((ch + NSLOT) * CR, WPR - CR)
            pltpu.async_copy(src_hbm.at[pl.ds(nofs, CR)], sidx[b], isem)
            pltpu.async_copy(dst_hbm.at[pl.ds(nofs, CR)], didx[b], isem)
        for h in range(2):
            off = offs[h]
            wo = wofs[h]

            @pl.when(off >= FRE)
            def _(h=h, wo=wo):
                woa = pl.multiple_of(wo, 128)
                pltpu.sync_copy(sbufs[h].at[pl.ds(0, FRE)],
                                psrc_hbm.at[h, wid, pl.ds(woa, FRE)])
                pltpu.sync_copy(dbufs[h].at[pl.ds(0, FRE)],
                                pdst_hbm.at[h, wid, pl.ds(woa, FRE)])
                for t in range(24):
                    rv = sbufs[h][pl.ds(FRE + t * LANES, LANES)]
                    sbufs[h][pl.ds(t * LANES, LANES)] = rv
                    rv2 = dbufs[h][pl.ds(FRE + t * LANES, LANES)]
                    dbufs[h][pl.ds(t * LANES, LANES)] = rv2

            offs[h] = jnp.where(off >= FRE, off - FRE, off)
            wofs[h] = jnp.where(off >= FRE, wo + FRE, wo)
        return offs[0], wofs[0], offs[1], wofs[1]

    z = jnp.int32(0)
    off0, wofs0, off1, wofs1 = lax.fori_loop(0, NCH // NSLOT, outer,
                                             (z, z, z, z))

    for b in range(NSLOT):
        pltpu.make_async_copy(src_hbm.at[pl.ds(0, CR)], sidx[b],
                              isem).wait()
        pltpu.make_async_copy(dst_hbm.at[pl.ds(0, CR)], didx[b],
                              isem).wait()

    for h in range(2):
        off = off0 if h == 0 else off1
        wo = wofs0 if h == 0 else wofs1
        total = wo + off
        unit = SJ * NSLOT * 128
        nunits = (total + unit - 1) // unit
        target = nunits * unit
        for t in range(48):
            sbufs[h][pl.ds(off + t * LANES, LANES)] = jnp.zeros(
                (LANES,), jnp.int32)
            dbufs[h][pl.ds(off + t * LANES, LANES)] = jnp.full(
                (LANES,), HALF, jnp.int32)

        def tail(r, wo_, h=h):
            ta = pl.multiple_of(wo_ + r * 128, 128)
            pltpu.sync_copy(sbufs[h].at[pl.ds(r * 128, 128)],
                            psrc_hbm.at[h, wid, pl.ds(ta, 128)])
            pltpu.sync_copy(dbufs[h].at[pl.ds(r * 128, 128)],
                            pdst_hbm.at[h, wid, pl.ds(ta, 128)])
            return wo_

        lax.fori_loop(0, (target - wo) // 128, tail, wo)
        cntw[pl.ds(0, LANES)] = jnp.full((LANES,), nunits, jnp.int32)
        pltpu.sync_copy(cntw, cnt_hbm.at[h, wid])


_part_call = functools.partial(
    pl.kernel,
    out_type=(
        jax.ShapeDtypeStruct((2, NW, CAPF), jnp.int32),
        jax.ShapeDtypeStruct((2, NW, CAPF), jnp.int32),
        jax.ShapeDtypeStruct((2, NW, LANES), jnp.int32),
    ),
    mesh=_sc_mesh,
    compiler_params=_sc_params,
    scratch_types=[
        pltpu.VMEM((CR, 128), jnp.int32) for _ in range(2 * NSLOT)
    ] + [pltpu.VMEM((BUFCAP,), jnp.int32) for _ in range(4)] + [
        pltpu.VMEM((LANES,), jnp.int32),
        pltpu.VMEM((LANES,), jnp.int32),
        pltpu.VMEM((LANES,), jnp.int32),
        pltpu.SemaphoreType.DMA,
    ],
)(_part_body)


def _edge_pass(gather, h_hbm, psrc_hbm, pdst_hbm, cnt_hbm, accum, sidx,
               didx, rows, dlocs, onesv, cntv, isem, gsem, ssem, cid, sid):
    for r in range(2):
        w = 2 * sid + r
        pltpu.sync_copy(cnt_hbm.at[cid, w], cntv)
        no = jnp.max(cntv[...])
        ngrp = no * NSLOT
        for b in range(NSLOT):
            pltpu.async_copy(psrc_hbm.at[cid, w, pl.ds(b * GRP, GRP)],
                             sidx[b], isem)
            pltpu.async_copy(pdst_hbm.at[cid, w, pl.ds(b * GRP, GRP)],
                             didx[b], isem)

        def outer(o, carry, w=w, ngrp=ngrp):
            for b in range(NSLOT):
                g = o * NSLOT + b

                @pl.when(o > 0)
                def _():
                    for j in range(SJ):
                        src = rows[b].at[j] if gather else onesv
                        pltpu.make_async_copy(src, accum.at[dlocs[b][j]],
                                              ssem).wait()

                pltpu.make_async_copy(psrc_hbm.at[cid, w, pl.ds(0, GRP)],
                                      sidx[b], isem).wait()
                pltpu.make_async_copy(pdst_hbm.at[cid, w, pl.ds(0, GRP)],
                                      didx[b], isem).wait()
                if gather:
                    gets = [pltpu.async_copy(
                        h_hbm.at[sidx[b].at[pl.ds(j * 128, 128)]],
                        rows[b].at[j], gsem) for j in range(SJ)]
                for j in range(SJ):
                    for k2 in range(128 // LANES):
                        dlocs[b][j][pl.ds(k2 * LANES, LANES)] = \
                            didx[b][pl.ds(j * 128 + k2 * LANES, LANES)]
                if gather:
                    for cp in gets:
                        cp.wait()
                for j in range(SJ):
                    src = rows[b].at[j] if gather else onesv
                    pltpu.async_copy(src, accum.at[dlocs[b][j]], ssem,
                                     add=True)
                nofs = pl.multiple_of(
                    jnp.minimum(g + NSLOT, ngrp - 1) * GRP, 128)
                pltpu.async_copy(psrc_hbm.at[cid, w, pl.ds(nofs, GRP)],
                                 sidx[b], isem)
                pltpu.async_copy(pdst_hbm.at[cid, w, pl.ds(nofs, GRP)],
                                 didx[b], isem)
            return carry

        lax.fori_loop(0, no, outer, 0)

        @pl.when(no > 0)
        def _():
            for b in range(NSLOT):
                for j in range(SJ):
                    src = rows[b].at[j] if gather else onesv
                    pltpu.make_async_copy(src, accum.at[dlocs[b][j]],
                                          ssem).wait()
        for b in range(NSLOT):
            pltpu.make_async_copy(psrc_hbm.at[cid, 0, pl.ds(0, GRP)],
                                  sidx[b], isem).wait()
            pltpu.make_async_copy(pdst_hbm.at[cid, 0, pl.ds(0, GRP)],
                                  didx[b], isem).wait()


def _flush(accum, out_hbm, cid, sid):
    base = cid * HALF

    @pl.when(sid < NS - 1)
    def _():
        pltpu.sync_copy(accum.at[pl.ds(sid * FL, FL)],
                        out_hbm.at[pl.ds(base + sid * FL, FL)])

    @pl.when(sid == NS - 1)
    def _():
        pltpu.sync_copy(accum.at[pl.ds((NS - 1) * FL, FLL)],
                        out_hbm.at[pl.ds(base + (NS - 1) * FL, FLL)])


def _agg_body(h_hbm, psrc_hbm, pdst_hbm, cnt_hbm, zeros_hbm, out_hbm,
              accum, sidx0, sidx1, didx0, didx1, rows0, rows1,
              dl00, dl01, dl02, dl10, dl11, dl12, cntv, isem, gsem, ssem):
    cid = lax.axis_index("c")
    sid = lax.axis_index("s")
    pltpu.sync_copy(zeros_hbm, accum.at[pl.ds(sid * FL, FL)])
    plsc.subcore_barrier()
    _edge_pass(True, h_hbm, psrc_hbm, pdst_hbm, cnt_hbm, accum,
               (sidx0, sidx1), (didx0, didx1), (rows0, rows1),
               ((dl00, dl01, dl02), (dl10, dl11, dl12)), None,
               cntv, isem, gsem, ssem, cid, sid)
    plsc.subcore_barrier()
    _flush(accum, out_hbm, cid, sid)


_agg_call = functools.partial(
    pl.kernel,
    out_type=jax.ShapeDtypeStruct((N, D), jnp.float32),
    mesh=_sc_mesh,
    compiler_params=_sc_params,
    scratch_types=[
        pltpu.VMEM_SHARED((ACC_ROWS, D), jnp.float32),
    ] + [pltpu.VMEM((GRP,), jnp.int32) for _ in range(2 * NSLOT)] + [
        pltpu.VMEM((SJ, 128, D), jnp.float32) for _ in range(NSLOT)
    ] + [pltpu.VMEM((128,), jnp.int32) for _ in range(NSLOT * SJ)] + [
        pltpu.VMEM((LANES,), jnp.int32),
        pltpu.SemaphoreType.DMA,
        pltpu.SemaphoreType.DMA,
        pltpu.SemaphoreType.DMA,
    ],
)(_agg_body)


def _deg_body(psrc_hbm, pdst_hbm, cnt_hbm, ones_hbm, zeros_hbm, out_hbm,
              accum, onesv, sidx0, sidx1, didx0, didx1,
              dl00, dl01, dl02, dl10, dl11, dl12, cntv, isem, gsem, ssem):
    cid = lax.axis_index("c")
    sid = lax.axis_index("s")
    pltpu.sync_copy(zeros_hbm, accum.at[pl.ds(sid * FL, FL)])
    pltpu.sync_copy(ones_hbm, onesv)
    plsc.subcore_barrier()
    _edge_pass(False, None, psrc_hbm, pdst_hbm, cnt_hbm, accum,
               (sidx0, sidx1), (didx0, didx1), None,
               ((dl00, dl01, dl02), (dl10, dl11, dl12)), onesv,
               cntv, isem, gsem, ssem, cid, sid)
    plsc.subcore_barrier()
    _flush(accum, out_hbm, cid, sid)


_deg_call = functools.partial(
    pl.kernel,
    out_type=jax.ShapeDtypeStruct((N, D), jnp.float32),
    mesh=_sc_mesh,
    compiler_params=_sc_params,
    scratch_types=[
        pltpu.VMEM_SHARED((ACC_ROWS, D), jnp.float32),
        pltpu.VMEM((128, D), jnp.float32),
    ] + [pltpu.VMEM((GRP,), jnp.int32) for _ in range(2 * NSLOT)] + [
        pltpu.VMEM((128,), jnp.int32) for _ in range(NSLOT * SJ)
    ] + [
        pltpu.VMEM((LANES,), jnp.int32),
        pltpu.SemaphoreType.DMA,
        pltpu.SemaphoreType.DMA,
        pltpu.SemaphoreType.DMA,
    ],
)(_deg_body)


BLK4 = 5000
NP4 = N * D // 128


def _dense_body(h_ref, agg_ref, deg_ref, w1_ref, b1_ref, w2_ref, b2_ref,
                gm_ref, bt_ref, o_ref):
    deg = jnp.maximum(deg_ref[...], 1.0)
    rst = h_ref[...] + agg_ref[...] / deg
    u = jnp.maximum(
        jnp.dot(rst, w1_ref[...], preferred_element_type=jnp.float32)
        + b1_ref[...], 0.0)
    y = jnp.dot(u, w2_ref[...], preferred_element_type=jnp.float32) \
        + b2_ref[...]
    o_ref[...] = gm_ref[...] * (jnp.maximum(y, 0.0) * BN_SCALE) + bt_ref[...]


_dense_call = pl.pallas_call(
    _dense_body,
    grid=(NP4 // BLK4,),
    in_specs=[
        pl.BlockSpec((BLK4, 128), lambda i: (i, 0)),
        pl.BlockSpec((BLK4, 128), lambda i: (i, 0)),
        pl.BlockSpec((BLK4, 128), lambda i: (i, 0)),
        pl.BlockSpec((128, 128), lambda i: (0, 0)),
        pl.BlockSpec((1, 128), lambda i: (0, 0)),
        pl.BlockSpec((128, 128), lambda i: (0, 0)),
        pl.BlockSpec((1, 128), lambda i: (0, 0)),
        pl.BlockSpec((1, 128), lambda i: (0, 0)),
        pl.BlockSpec((1, 128), lambda i: (0, 0)),
    ],
    out_specs=pl.BlockSpec((BLK4, 128), lambda i: (i, 0)),
    out_shape=jax.ShapeDtypeStruct((NP4, 128), jnp.float32),
)


def _gather_body(idx_ref, h_ref, o_ref):
    o_ref[...] = h_ref[...]


_gather_call = pl.pallas_call(
    _gather_body,
    grid_spec=pltpu.PrefetchScalarGridSpec(
        num_scalar_prefetch=1,
        grid=(NB,),
        in_specs=[pl.BlockSpec((1, 1, D), lambda i, idx: (idx[i], 0, 0))],
        out_specs=pl.BlockSpec((1, 1, D), lambda i, idx: (i, 0, 0)),
    ),
    out_shape=jax.ShapeDtypeStruct((NB, 1, D), jnp.float32),
)


def _head_body(g_ref, w1_ref, b1_ref, w2_ref, b2_ref, o_ref):
    g1 = jnp.maximum(
        jnp.dot(g_ref[...], w1_ref[...], preferred_element_type=jnp.float32)
        + b1_ref[...], 0.0)
    logits = jnp.dot(g1, w2_ref[...], preferred_element_type=jnp.float32) \
        + b2_ref[...]
    m = jnp.max(logits, axis=-1, keepdims=True)
    lse = jnp.log(jnp.sum(jnp.exp(logits - m), axis=-1, keepdims=True)) + m
    o_ref[...] = logits - lse


_head_call = pl.pallas_call(
    _head_body,
    out_shape=jax.ShapeDtypeStruct((NB, NCLS), jnp.float32),
)


def kernel(x, edge_index, batch_num_nodes, W1, b1, W2, b2, gamma, beta,
           fc1_w, fc1_b, fc2_w, fc2_b):
    src = edge_index[0]
    dst = edge_index[1]
    pad = EPAD - E
    src2 = jnp.concatenate(
        [src, jnp.zeros((pad,), jnp.int32)]).reshape(ROWS, 128)
    dst2 = jnp.concatenate(
        [dst, jnp.full((pad,), -1, jnp.int32)]).reshape(ROWS, 128)
    zeros32 = jnp.zeros((FL, D), jnp.float32)
    ones32 = jnp.ones((128, D), jnp.float32)

    psrc, pdst, cnt = _part_call(src2, dst2)
    degf = _deg_call(psrc, pdst, cnt, ones32, zeros32)
    deg4 = degf.reshape(NP4, 128)
    eye4 = jnp.eye(4, dtype=jnp.float32)

    h = x
    for i in range(NL):
        agg = _agg_call(h, psrc, pdst, cnt, zeros32)
        h4 = _dense_call(
            h.reshape(NP4, 128), agg.reshape(NP4, 128), deg4,
            jnp.kron(eye4, W1[i]), jnp.tile(b1[i], 4).reshape(1, 128),
            jnp.kron(eye4, W2[i]), jnp.tile(b2[i], 4).reshape(1, 128),
            jnp.tile(gamma[i], 4).reshape(1, 128),
            jnp.tile(beta[i], 4).reshape(1, 128))
        h = h4.reshape(N, D)

    idx = (jnp.cumsum(batch_num_nodes) - 1).astype(jnp.int32)
    g = _gather_call(idx, h.reshape(N, 1, D)).reshape(NB, D)
    return _head_call(g, fc1_w, fc1_b.reshape(1, D), fc2_w,
                      fc2_b.reshape(1, NCLS))


# speedup vs baseline: 3.0280x; 1.0181x over previous
import functools
import math

import jax
import jax.numpy as jnp
from jax import lax
from jax.experimental import pallas as pl
from jax.experimental.pallas import tpu as pltpu
from jax.experimental.pallas import tpu_sc as plsc

N = 100000
D = 32
E = 1600000
NL = 5
NB = 100
NCLS = 10
HALF = N // 2
NC = 2
NS = 16
LANES = 16

NW = NC * NS
WPR = 396
ROWS = NW * WPR
EPAD = ROWS * 128
EROWS = E // 128
CR = 2
NCH = WPR // CR
CAPR = 396
CAPF = CAPR * 128
FRE = 4096
BUFCAP = 5376

SJ = 3
NSLOT = 2
GRP = SJ * 128

FL = 3128
FLL = HALF - (NS - 1) * FL
ACC_ROWS = NS * FL
BN_SCALE = 1.0 / math.sqrt(1.0 + 1e-5)

_sc_mesh = plsc.VectorSubcoreMesh(core_axis_name="c", subcore_axis_name="s")
_sc_params = pltpu.CompilerParams(use_tc_tiling_on_sc=False,
                                  needs_layout_passes=False)


def _part_body(edge_hbm, psrc_hbm, pdst_hbm, cnt_hbm,
               sidx0, sidx1, didx0, didx1,
               sbuf0, dbuf0, sbuf1, dbuf1, stgs, stgd, cntw, isem):
    sidx = (sidx0, sidx1)
    didx = (didx0, didx1)
    sbufs = (sbuf0, sbuf1)
    dbufs = (dbuf0, dbuf1)
    cid = lax.axis_index("c")
    sid = lax.axis_index("s")
    wid = cid * NS + sid
    inbase = wid * WPR

    lane = lax.iota(jnp.int32, LANES)
    for b in range(NSLOT):
        ofs = jnp.minimum(inbase + b * CR, EROWS - CR)
        pltpu.async_copy(edge_hbm.at[0, pl.ds(ofs, CR)], sidx[b], isem)
        pltpu.async_copy(edge_hbm.at[1, pl.ds(ofs, CR)], didx[b], isem)

    def outer(o, st):
        off0, wofs0, off1, wofs1 = st
        offs = [off0, off1]
        wofs = [wofs0, wofs1]
        for b in range(NSLOT):
            ch = o * NSLOT + b
            pltpu.make_async_copy(edge_hbm.at[0, pl.ds(0, CR)], sidx[b],
                                  isem).wait()
            pltpu.make_async_copy(edge_hbm.at[1, pl.ds(0, CR)], didx[b],
                                  isem).wait()
            for j in range(CR):
                for k2 in range(128 // LANES):
                    s_v = sidx[b][j, pl.ds(k2 * LANES, LANES)]
                    d_v = didx[b][j, pl.ds(k2 * LANES, LANES)]
                    pos = ((inbase + ch * CR + j) * 128 + k2 * LANES) + lane
                    valid = pos < E
                    for h in range(2):
                        if h == 0:
                            m = valid & (d_v < HALF)
                            dl = d_v
                        else:
                            m = valid & (d_v >= HALF)
                            dl = d_v - HALF
                        plsc.store_compressed(stgs.at[pl.ds(0, LANES)],
                                              s_v, mask=m)
                        plsc.store_compressed(stgd.at[pl.ds(0, LANES)],
                                              dl, mask=m)
                        c = jnp.max(plsc.all_reduce_population_count(m))
                        sbufs[h][pl.ds(offs[h], LANES)] = stgs[...]
                        dbufs[h][pl.ds(offs[h], LANES)] = stgd[...]
                        offs[h] = offs[h] + c
            nofs = jnp.minimum(inbase + jnp.minimum((ch + NSLOT) * CR,
                                                    WPR - CR), EROWS - CR)
            pltpu.async_copy(edge_hbm.at[0, pl.ds(nofs, CR)], sidx[b],
                             isem)
            pltpu.async_copy(edge_hbm.at[1, pl.ds(nofs, CR)], didx[b],
                             isem)
        for h in range(2):
            off = offs[h]
            wo = wofs[h]

            @pl.when(off >= FRE)
            def _(h=h, wo=wo):
                woa = pl.multiple_of(wo, 128)
                pltpu.sync_copy(sbufs[h].at[pl.ds(0, FRE)],
                                psrc_hbm.at[h, wid, pl.ds(woa, FRE)])
                pltpu.sync_copy(dbufs[h].at[pl.ds(0, FRE)],
                                pdst_hbm.at[h, wid, pl.ds(woa, FRE)])
                for t in range(24):
                    rv = sbufs[h][pl.ds(FRE + t * LANES, LANES)]
                    sbufs[h][pl.ds(t * LANES, LANES)] = rv
                    rv2 = dbufs[h][pl.ds(FRE + t * LANES, LANES)]
                    dbufs[h][pl.ds(t * LANES, LANES)] = rv2

            offs[h] = jnp.where(off >= FRE, off - FRE, off)
            wofs[h] = jnp.where(off >= FRE, wo + FRE, wo)
        return offs[0], wofs[0], offs[1], wofs[1]

    z = jnp.int32(0)
    off0, wofs0, off1, wofs1 = lax.fori_loop(0, NCH // NSLOT, outer,
                                             (z, z, z, z))

    for b in range(NSLOT):
        pltpu.make_async_copy(edge_hbm.at[0, pl.ds(0, CR)], sidx[b],
                              isem).wait()
        pltpu.make_async_copy(edge_hbm.at[1, pl.ds(0, CR)], didx[b],
                              isem).wait()

    for h in range(2):
        off = off0 if h == 0 else off1
        wo = wofs0 if h == 0 else wofs1
        total = wo + off
        unit = SJ * NSLOT * 128
        nunits = (total + unit - 1) // unit
        target = nunits * unit
        for t in range(48):
            sbufs[h][pl.ds(off + t * LANES, LANES)] = jnp.zeros(
                (LANES,), jnp.int32)
            dbufs[h][pl.ds(off + t * LANES, LANES)] = jnp.full(
                (LANES,), HALF, jnp.int32)

        def tail(r, wo_, h=h):
            ta = pl.multiple_of(wo_ + r * 128, 128)
            pltpu.sync_copy(sbufs[h].at[pl.ds(r * 128, 128)],
                            psrc_hbm.at[h, wid, pl.ds(ta, 128)])
            pltpu.sync_copy(dbufs[h].at[pl.ds(r * 128, 128)],
                            pdst_hbm.at[h, wid, pl.ds(ta, 128)])
            return wo_

        lax.fori_loop(0, (target - wo) // 128, tail, wo)
        cntw[pl.ds(0, LANES)] = jnp.full((LANES,), nunits, jnp.int32)
        pltpu.sync_copy(cntw, cnt_hbm.at[h, wid])


_part_call = functools.partial(
    pl.kernel,
    out_type=(
        jax.ShapeDtypeStruct((2, NW, CAPF), jnp.int32),
        jax.ShapeDtypeStruct((2, NW, CAPF), jnp.int32),
        jax.ShapeDtypeStruct((2, NW, LANES), jnp.int32),
    ),
    mesh=_sc_mesh,
    compiler_params=_sc_params,
    scratch_types=[
        pltpu.VMEM((CR, 128), jnp.int32) for _ in range(2 * NSLOT)
    ] + [pltpu.VMEM((BUFCAP,), jnp.int32) for _ in range(4)] + [
        pltpu.VMEM((LANES,), jnp.int32),
        pltpu.VMEM((LANES,), jnp.int32),
        pltpu.VMEM((LANES,), jnp.int32),
        pltpu.SemaphoreType.DMA,
    ],
)(_part_body)


def _edge_pass(gather, h_hbm, psrc_hbm, pdst_hbm, cnt_hbm, accum, sidx,
               didx, rows, dlocs, onesv, cntv, isem, gsem, ssem, cid, sid):
    for r in range(2):
        w = 2 * sid + r
        pltpu.sync_copy(cnt_hbm.at[cid, w], cntv)
        no = jnp.max(cntv[...])
        ngrp = no * NSLOT
        for b in range(NSLOT):
            pltpu.async_copy(psrc_hbm.at[cid, w, pl.ds(b * GRP, GRP)],
                             sidx[b], isem)
            pltpu.async_copy(pdst_hbm.at[cid, w, pl.ds(b * GRP, GRP)],
                             didx[b], isem)

        def outer(o, carry, w=w, ngrp=ngrp):
            for b in range(NSLOT):
                g = o * NSLOT + b

                @pl.when(o > 0)
                def _():
                    for j in range(SJ):
                        src = rows[b].at[j] if gather else onesv
                        pltpu.make_async_copy(src, accum.at[dlocs[b][j]],
                                              ssem).wait()

                pltpu.make_async_copy(psrc_hbm.at[cid, w, pl.ds(0, GRP)],
                                      sidx[b], isem).wait()
                pltpu.make_async_copy(pdst_hbm.at[cid, w, pl.ds(0, GRP)],
                                      didx[b], isem).wait()
                if gather:
                    gets = [pltpu.async_copy(
                        h_hbm.at[sidx[b].at[pl.ds(j * 128, 128)]],
                        rows[b].at[j], gsem) for j in range(SJ)]
                for j in range(SJ):
                    for k2 in range(128 // LANES):
                        dlocs[b][j][pl.ds(k2 * LANES, LANES)] = \
                            didx[b][pl.ds(j * 128 + k2 * LANES, LANES)]
                if gather:
                    for cp in gets:
                        cp.wait()
                for j in range(SJ):
                    src = rows[b].at[j] if gather else onesv
                    pltpu.async_copy(src, accum.at[dlocs[b][j]], ssem,
                                     add=True)
                nofs = pl.multiple_of(
                    jnp.minimum(g + NSLOT, ngrp - 1) * GRP, 128)
                pltpu.async_copy(psrc_hbm.at[cid, w, pl.ds(nofs, GRP)],
                                 sidx[b], isem)
                pltpu.async_copy(pdst_hbm.at[cid, w, pl.ds(nofs, GRP)],
                                 didx[b], isem)
            return carry

        lax.fori_loop(0, no, outer, 0)

        @pl.when(no > 0)
        def _():
            for b in range(NSLOT):
                for j in range(SJ):
                    src = rows[b].at[j] if gather else onesv
                    pltpu.make_async_copy(src, accum.at[dlocs[b][j]],
                                          ssem).wait()
        for b in range(NSLOT):
            pltpu.make_async_copy(psrc_hbm.at[cid, 0, pl.ds(0, GRP)],
                                  sidx[b], isem).wait()
            pltpu.make_async_copy(pdst_hbm.at[cid, 0, pl.ds(0, GRP)],
                                  didx[b], isem).wait()


def _flush(accum, out_hbm, cid, sid):
    base = cid * HALF

    @pl.when(sid < NS - 1)
    def _():
        pltpu.sync_copy(accum.at[pl.ds(sid * FL, FL)],
                        out_hbm.at[pl.ds(base + sid * FL, FL)])

    @pl.when(sid == NS - 1)
    def _():
        pltpu.sync_copy(accum.at[pl.ds((NS - 1) * FL, FLL)],
                        out_hbm.at[pl.ds(base + (NS - 1) * FL, FLL)])


def _agg_body(h_hbm, psrc_hbm, pdst_hbm, cnt_hbm, zeros_hbm, out_hbm,
              accum, sidx0, sidx1, didx0, didx1, rows0, rows1,
              dl00, dl01, dl02, dl10, dl11, dl12, cntv, isem, gsem, ssem):
    cid = lax.axis_index("c")
    sid = lax.axis_index("s")
    pltpu.sync_copy(zeros_hbm, accum.at[pl.ds(sid * FL, FL)])
    plsc.subcore_barrier()
    _edge_pass(True, h_hbm, psrc_hbm, pdst_hbm, cnt_hbm, accum,
               (sidx0, sidx1), (didx0, didx1), (rows0, rows1),
               ((dl00, dl01, dl02), (dl10, dl11, dl12)), None,
               cntv, isem, gsem, ssem, cid, sid)
    plsc.subcore_barrier()
    _flush(accum, out_hbm, cid, sid)


_agg_call = functools.partial(
    pl.kernel,
    out_type=jax.ShapeDtypeStruct((N, D), jnp.float32),
    mesh=_sc_mesh,
    compiler_params=_sc_params,
    scratch_types=[
        pltpu.VMEM_SHARED((ACC_ROWS, D), jnp.float32),
    ] + [pltpu.VMEM((GRP,), jnp.int32) for _ in range(2 * NSLOT)] + [
        pltpu.VMEM((SJ, 128, D), jnp.float32) for _ in range(NSLOT)
    ] + [pltpu.VMEM((128,), jnp.int32) for _ in range(NSLOT * SJ)] + [
        pltpu.VMEM((LANES,), jnp.int32),
        pltpu.SemaphoreType.DMA,
        pltpu.SemaphoreType.DMA,
        pltpu.SemaphoreType.DMA,
    ],
)(_agg_body)


def _deg_body(psrc_hbm, pdst_hbm, cnt_hbm, ones_hbm, zeros_hbm, out_hbm,
              accum, onesv, sidx0, sidx1, didx0, didx1,
              dl00, dl01, dl02, dl10, dl11, dl12, cntv, isem, gsem, ssem):
    cid = lax.axis_index("c")
    sid = lax.axis_index("s")
    pltpu.sync_copy(zeros_hbm, accum.at[pl.ds(sid * FL, FL)])
    pltpu.sync_copy(ones_hbm, onesv)
    plsc.subcore_barrier()
    _edge_pass(False, None, psrc_hbm, pdst_hbm, cnt_hbm, accum,
               (sidx0, sidx1), (didx0, didx1), None,
               ((dl00, dl01, dl02), (dl10, dl11, dl12)), onesv,
               cntv, isem, gsem, ssem, cid, sid)
    plsc.subcore_barrier()
    _flush(accum, out_hbm, cid, sid)


_deg_call = functools.partial(
    pl.kernel,
    out_type=jax.ShapeDtypeStruct((N, D), jnp.float32),
    mesh=_sc_mesh,
    compiler_params=_sc_params,
    scratch_types=[
        pltpu.VMEM_SHARED((ACC_ROWS, D), jnp.float32),
        pltpu.VMEM((128, D), jnp.float32),
    ] + [pltpu.VMEM((GRP,), jnp.int32) for _ in range(2 * NSLOT)] + [
        pltpu.VMEM((128,), jnp.int32) for _ in range(NSLOT * SJ)
    ] + [
        pltpu.VMEM((LANES,), jnp.int32),
        pltpu.SemaphoreType.DMA,
        pltpu.SemaphoreType.DMA,
        pltpu.SemaphoreType.DMA,
    ],
)(_deg_body)


BLK4 = 5000
NP4 = N * D // 128


def _dense_body(h_ref, agg_ref, deg_ref, w1_ref, b1_ref, w2_ref, b2_ref,
                gm_ref, bt_ref, o_ref):
    deg = jnp.maximum(deg_ref[...], 1.0)
    rst = h_ref[...] + agg_ref[...] / deg
    u = jnp.maximum(
        jnp.dot(rst, w1_ref[...], preferred_element_type=jnp.float32)
        + b1_ref[...], 0.0)
    y = jnp.dot(u, w2_ref[...], preferred_element_type=jnp.float32) \
        + b2_ref[...]
    o_ref[...] = gm_ref[...] * (jnp.maximum(y, 0.0) * BN_SCALE) + bt_ref[...]


_dense_call = pl.pallas_call(
    _dense_body,
    grid=(NP4 // BLK4,),
    in_specs=[
        pl.BlockSpec((BLK4, 128), lambda i: (i, 0)),
        pl.BlockSpec((BLK4, 128), lambda i: (i, 0)),
        pl.BlockSpec((BLK4, 128), lambda i: (i, 0)),
        pl.BlockSpec((128, 128), lambda i: (0, 0)),
        pl.BlockSpec((1, 128), lambda i: (0, 0)),
        pl.BlockSpec((128, 128), lambda i: (0, 0)),
        pl.BlockSpec((1, 128), lambda i: (0, 0)),
        pl.BlockSpec((1, 128), lambda i: (0, 0)),
        pl.BlockSpec((1, 128), lambda i: (0, 0)),
    ],
    out_specs=pl.BlockSpec((BLK4, 128), lambda i: (i, 0)),
    out_shape=jax.ShapeDtypeStruct((NP4, 128), jnp.float32),
)


def _gather_body(idx_ref, h_ref, o_ref):
    o_ref[...] = h_ref[...]


_gather_call = pl.pallas_call(
    _gather_body,
    grid_spec=pltpu.PrefetchScalarGridSpec(
        num_scalar_prefetch=1,
        grid=(NB,),
        in_specs=[pl.BlockSpec((1, 1, 128), lambda i, idx: (idx[i], 0, 0))],
        out_specs=pl.BlockSpec((1, 1, 128), lambda i, idx: (i, 0, 0)),
    ),
    out_shape=jax.ShapeDtypeStruct((NB, 1, 128), jnp.float32),
)


def _head_body(g_ref, sel_ref, w1_ref, b1_ref, w2_ref, b2_ref, o_ref):
    g128 = g_ref[...]
    sel = sel_ref[...]
    g = jnp.zeros((NB, D), jnp.float32)
    for q in range(4):
        g = g + jnp.where(sel == q, g128[:, q * D:(q + 1) * D], 0.0)
    g1 = jnp.maximum(
        jnp.dot(g, w1_ref[...], preferred_element_type=jnp.float32)
        + b1_ref[...], 0.0)
    logits = jnp.dot(g1, w2_ref[...], preferred_element_type=jnp.float32) \
        + b2_ref[...]
    m = jnp.max(logits, axis=-1, keepdims=True)
    lse = jnp.log(jnp.sum(jnp.exp(logits - m), axis=-1, keepdims=True)) + m
    o_ref[...] = logits - lse


_head_call = pl.pallas_call(
    _head_body,
    out_shape=jax.ShapeDtypeStruct((NB, NCLS), jnp.float32),
)


def kernel(x, edge_index, batch_num_nodes, W1, b1, W2, b2, gamma, beta,
           fc1_w, fc1_b, fc2_w, fc2_b):
    edges = edge_index.reshape(2, EROWS, 128)
    zeros32 = jnp.zeros((FL, D), jnp.float32)
    ones32 = jnp.ones((128, D), jnp.float32)

    psrc, pdst, cnt = _part_call(edges)
    degf = _deg_call(psrc, pdst, cnt, ones32, zeros32)
    deg4 = degf.reshape(NP4, 128)
    eye4 = jnp.eye(4, dtype=jnp.float32)

    h4 = x.reshape(NP4, 128)
    h = h4.reshape(N, D)
    for i in range(NL):
        agg = _agg_call(h, psrc, pdst, cnt, zeros32)
        h4 = _dense_call(
            h.reshape(NP4, 128), agg.reshape(NP4, 128), deg4,
            jnp.kron(eye4, W1[i]), jnp.tile(b1[i], 4).reshape(1, 128),
            jnp.kron(eye4, W2[i]), jnp.tile(b2[i], 4).reshape(1, 128),
            jnp.tile(gamma[i], 4).reshape(1, 128),
            jnp.tile(beta[i], 4).reshape(1, 128))
        h = h4.reshape(N, D)

    tri = jnp.triu(jnp.ones((NB, NB), jnp.float32))
    idx = (jnp.dot(batch_num_nodes.astype(jnp.float32), tri)
           .astype(jnp.int32) - 1)
    g128 = _gather_call((idx // 4).astype(jnp.int32),
                        h4.reshape(NP4, 1, 128)).reshape(NB, 128)
    sel = (idx % 4).astype(jnp.int32).reshape(NB, 1)
    return _head_call(g128, sel, fc1_w, fc1_b.reshape(1, D), fc2_w,
                      fc2_b.reshape(1, NCLS))
